# Initial kernel scaffold; baseline (speedup 1.0000x reference)
import jax
import jax.numpy as jnp
from jax.experimental import pallas as pl


def kernel(task_id, mesa_parameters):
    raise NotImplementedError("write your pallas kernel here")


# speedup vs baseline: 1.4101x; 1.4101x over previous
import functools

import jax
import jax.numpy as jnp
from jax import lax
from jax.experimental import pallas as pl
from jax.experimental.pallas import tpu as pltpu
from jax.experimental.pallas import tpu_sc as plsc

D = 64
V = 100000
B = 16384
NC, NS, L = 2, 16, 16
NW = NC * NS
ROWS_PER_W = D // NW
CHUNK = 8192


def _gather_kernel(idx_hbm, table_hbm, out_hbm, idx_v, row_v, out_v):
    wid = lax.axis_index("s") * NC + lax.axis_index("c")
    pltpu.sync_copy(idx_hbm, idx_v)
    for r in range(ROWS_PER_W):
        row = wid * ROWS_PER_W + r
        pltpu.sync_copy(table_hbm.at[row], row_v)
        for c in range(B // CHUNK):
            def body(k, carry):
                iv = idx_v[pl.ds(c * CHUNK + k * L, L)]
                out_v[pl.ds(k * L, L)] = plsc.load_gather(row_v, [iv])
                return carry
            lax.fori_loop(0, CHUNK // L, body, 0)
            pltpu.sync_copy(out_v, out_hbm.at[row, pl.ds(c * CHUNK, CHUNK)])


@jax.jit
def _run(task_id, mesa_parameters):
    mesh = plsc.VectorSubcoreMesh(core_axis_name="c", subcore_axis_name="s")
    return pl.kernel(
        _gather_kernel,
        out_type=jax.ShapeDtypeStruct((D, B), jnp.float32),
        mesh=mesh,
        scratch_types=[
            pltpu.VMEM((B,), jnp.int32),
            pltpu.VMEM((V,), jnp.float32),
            pltpu.VMEM((CHUNK,), jnp.float32),
        ],
        compiler_params=pltpu.CompilerParams(needs_layout_passes=False),
    )(task_id, mesa_parameters)


def kernel(task_id, mesa_parameters):
    return _run(task_id.astype(jnp.int32), mesa_parameters)


# speedup vs baseline: 1.9635x; 1.3925x over previous
import jax
import jax.numpy as jnp
from jax import lax
from jax.experimental import pallas as pl
from jax.experimental.pallas import tpu as pltpu
from jax.experimental.pallas import tpu_sc as plsc

D = 64
V = 100000
B = 16384
NC, NS, L = 2, 16, 16
NW = NC * NS
ROWS_PER_W = D // NW
CHUNK = 4096
NCHUNK = B // CHUNK


def _gather_kernel(idx_hbm, table_hbm, out_hbm,
                   idx_v, row_v, out0_v, out1_v, sem_idx, sem_row, sem_o0, sem_o1):
    wid = lax.axis_index("s") * NC + lax.axis_index("c")
    base_row = wid * ROWS_PER_W
    cp_idx = pltpu.async_copy(idx_hbm, idx_v, sem_idx)
    cp_row = pltpu.async_copy(table_hbm.at[base_row], row_v, sem_row)
    cp_idx.wait()
    cp_row.wait()

    out_cps = [None, None]
    out_sems = [sem_o0, sem_o1]
    out_bufs = [out0_v, out1_v]
    for r in range(ROWS_PER_W):
        row = base_row + r
        for c in range(NCHUNK):
            t = (r * NCHUNK + c) % 2
            if out_cps[t] is not None:
                out_cps[t].wait()
            buf = out_bufs[t]

            @plsc.parallel_loop(0, CHUNK, step=L, unroll=8)
            def body(k):
                iv = idx_v[pl.ds(c * CHUNK + k, L)]
                buf[pl.ds(k, L)] = plsc.load_gather(row_v, [iv])

            out_cps[t] = pltpu.async_copy(
                buf, out_hbm.at[row, pl.ds(c * CHUNK, CHUNK)], out_sems[t])
        if r + 1 < ROWS_PER_W:
            pltpu.async_copy(table_hbm.at[base_row + r + 1], row_v, sem_row).wait()
    for cp in out_cps:
        cp.wait()


@jax.jit
def _run(task_id, mesa_parameters):
    mesh = plsc.VectorSubcoreMesh(core_axis_name="c", subcore_axis_name="s")
    return pl.kernel(
        _gather_kernel,
        out_type=jax.ShapeDtypeStruct((D, B), jnp.float32),
        mesh=mesh,
        scratch_types=[
            pltpu.VMEM((B,), jnp.int32),
            pltpu.VMEM((V,), jnp.float32),
            pltpu.VMEM((CHUNK,), jnp.float32),
            pltpu.VMEM((CHUNK,), jnp.float32),
            pltpu.SemaphoreType.DMA,
            pltpu.SemaphoreType.DMA,
            pltpu.SemaphoreType.DMA,
            pltpu.SemaphoreType.DMA,
        ],
        compiler_params=pltpu.CompilerParams(needs_layout_passes=False),
    )(task_id, mesa_parameters)


def kernel(task_id, mesa_parameters):
    return _run(task_id.astype(jnp.int32), mesa_parameters)


# speedup vs baseline: 1.9671x; 1.0019x over previous
import jax
import jax.numpy as jnp
from jax import lax
from jax.experimental import pallas as pl
from jax.experimental.pallas import tpu as pltpu
from jax.experimental.pallas import tpu_sc as plsc

D = 64
V = 100000
B = 16384
NC, NS, L = 2, 16, 16
NW = NC * NS
ROWS_PER_W = D // NW
CHUNK = 4096
NCHUNK = B // CHUNK


def _gather_kernel(idx_hbm, table_hbm, out_hbm,
                   idx_v, row_v, out0_v, out1_v, sem_idx, sem_row, sem_o0, sem_o1):
    wid = lax.axis_index("s") * NC + lax.axis_index("c")
    base_row = wid * ROWS_PER_W
    cp_idx = pltpu.async_copy(idx_hbm, idx_v, sem_idx)
    cp_row = pltpu.async_copy(table_hbm.at[base_row], row_v, sem_row)
    cp_idx.wait()
    cp_row.wait()

    out_cps = [None, None]
    out_sems = [sem_o0, sem_o1]
    out_bufs = [out0_v, out1_v]
    for r in range(ROWS_PER_W):
        row = base_row + r
        for c in range(NCHUNK):
            t = (r * NCHUNK + c) % 2
            if out_cps[t] is not None:
                out_cps[t].wait()
            buf = out_bufs[t]

            @plsc.parallel_loop(0, CHUNK, step=L, unroll=8)
            def body(k):
                iv = idx_v[pl.ds(c * CHUNK + k, L)]
                buf[pl.ds(k, L)] = plsc.load_gather(row_v, [iv])

            out_cps[t] = pltpu.async_copy(
                buf, out_hbm.at[row, pl.ds(c * CHUNK, CHUNK)], out_sems[t])
        if r + 1 < ROWS_PER_W:
            pltpu.async_copy(table_hbm.at[base_row + r + 1], row_v, sem_row).wait()
    for cp in out_cps:
        cp.wait()


@jax.jit
def _run(task_id, mesa_parameters):
    mesh = plsc.VectorSubcoreMesh(core_axis_name="c", subcore_axis_name="s")
    return pl.kernel(
        _gather_kernel,
        out_type=jax.ShapeDtypeStruct((D, B), jnp.float32),
        mesh=mesh,
        scratch_types=[
            pltpu.VMEM((B,), jnp.int32),
            pltpu.VMEM((V,), jnp.float32),
            pltpu.VMEM((CHUNK,), jnp.float32),
            pltpu.VMEM((CHUNK,), jnp.float32),
            pltpu.SemaphoreType.DMA,
            pltpu.SemaphoreType.DMA,
            pltpu.SemaphoreType.DMA,
            pltpu.SemaphoreType.DMA,
        ],
        compiler_params=pltpu.CompilerParams(
            needs_layout_passes=False, skip_device_barrier=True),
    )(task_id, mesa_parameters)


def kernel(task_id, mesa_parameters):
    return _run(task_id.astype(jnp.int32), mesa_parameters)
